# Initial kernel scaffold; baseline (speedup 1.0000x reference)
import jax
import jax.numpy as jnp
from jax.experimental import pallas as pl


def kernel(x, exc, g_exc_x_w, g_exc_x_b, ln_e_x_g, ln_e_x_b, g_exc_e_w, g_exc_e_b, ln_e_e_g, ln_e_e_b, w_exc_x_w, w_exc_x_b, w_exc_ee_w, w_exc_ee_b, ln_out_e_g, ln_out_e_b):
    raise NotImplementedError("write your pallas kernel here")


# speedup vs baseline: 11.3400x; 11.3400x over previous
import functools

import jax
import jax.numpy as jnp
from jax.experimental import pallas as pl
from jax.experimental.pallas import tpu as pltpu

_F = 7
_P = (_F - 1) // 2
_EPS = 1e-5


def _gn_coeffs(v, gamma_row, beta_row, inv_n):
    s = jnp.sum(jnp.sum(v, axis=0, keepdims=True), axis=1, keepdims=True)
    sq = jnp.sum(jnp.sum(v * v, axis=0, keepdims=True), axis=1, keepdims=True)
    mu = s * inv_n
    var = sq * inv_n - mu * mu
    r = jax.lax.rsqrt(var + _EPS)
    scale = r * gamma_row
    offset = beta_row - mu * scale
    return scale, offset


def _cell_kernel(
    H, W, Ch, MT,
    xe_ref,
    e_ref,
    w3_ref,
    b3_ref,
    w7_ref,
    g1g_ref, g1b_ref,
    g2g_ref, g2b_ref,
    bee_ref,
    g3g_ref, g3b_ref,
    out_ref,
    ew_ref,
    big_ref,
    pre_ref,
):
    HW = H * W
    PW = _P * W
    inv_n = 1.0 / float(HW * Ch)

    big_ref[...] = (
        jnp.dot(xe_ref[0], w3_ref[...], preferred_element_type=jnp.float32)
        + b3_ref[...]
    )

    ew_ref[0:PW, :] = jnp.zeros((PW, _F * Ch), jnp.bfloat16)
    ew_ref[PW + HW:, :] = jnp.zeros((PW, _F * Ch), jnp.bfloat16)

    e2 = e_ref[0]
    pcol = jax.lax.broadcasted_iota(jnp.int32, (HW, Ch), 0) & (W - 1)
    for kw in range(_F):
        d = kw - _P
        rolled = pltpu.roll(e2, (-d) % HW, axis=0) if d else e2
        if d > 0:
            blk = jnp.where(pcol <= (W - 1 - d), rolled, 0.0)
        elif d < 0:
            blk = jnp.where(pcol >= (-d), rolled, 0.0)
        else:
            blk = rolled
        ew_ref[PW:PW + HW, kw * Ch:(kw + 1) * Ch] = blk.astype(jnp.bfloat16)

    c1, o1 = _gn_coeffs(big_ref[:, 0:Ch], g1g_ref[...], g1b_ref[...], inv_n)
    c2, o2 = _gn_coeffs(big_ref[:, Ch:2 * Ch], g2g_ref[...], g2b_ref[...], inv_n)
    o12 = o1 + o2

    bee = bee_ref[...]
    for m0 in range(0, HW, MT):
        acc = jnp.dot(ew_ref[m0:m0 + MT, :], w7_ref[0],
                      preferred_element_type=jnp.float32)
        for kh in range(1, _F):
            acc = acc + jnp.dot(ew_ref[m0 + kh * W:m0 + kh * W + MT, :],
                                w7_ref[kh], preferred_element_type=jnp.float32)
        cand = jnp.maximum(acc + big_ref[m0:m0 + MT, 2 * Ch:3 * Ch] + bee, 0.0)
        g = jax.nn.sigmoid(big_ref[m0:m0 + MT, 0:Ch] * c1
                           + big_ref[m0:m0 + MT, Ch:2 * Ch] * c2 + o12)
        et = e_ref[0, m0:m0 + MT, :]
        pre_ref[m0:m0 + MT, :] = g * cand + et - g * et

    c3, o3 = _gn_coeffs(pre_ref[...], g3g_ref[...], g3b_ref[...], inv_n)
    out_ref[0] = jnp.maximum(pre_ref[...] * c3 + o3, 0.0)


@jax.jit
def kernel(x, exc, g_exc_x_w, g_exc_x_b, ln_e_x_g, ln_e_x_b,
           g_exc_e_w, g_exc_e_b, ln_e_e_g, ln_e_e_b,
           w_exc_x_w, w_exc_x_b, w_exc_ee_w, w_exc_ee_b,
           ln_out_e_g, ln_out_e_b):
    N, Cin, H, W = x.shape
    Ch = exc.shape[1]
    HW = H * W
    MT = 256
    K3 = Cin + Ch

    x_t = jnp.transpose(x.reshape(N, Cin, HW), (0, 2, 1))
    e_t = jnp.transpose(exc.reshape(N, Ch, HW), (0, 2, 1))
    xe = jnp.concatenate([x_t, e_t], axis=2).astype(jnp.bfloat16)

    wgx = g_exc_x_w.reshape(Ch, Cin).T
    wge = g_exc_e_w.reshape(Ch, Ch).T
    wwx = w_exc_x_w.reshape(Ch, Cin).T
    z_ec = jnp.zeros((Ch, Ch), jnp.float32)
    z_xc = jnp.zeros((Cin, Ch), jnp.float32)
    w3 = jnp.concatenate([
        jnp.concatenate([wgx, z_xc, wwx], axis=1),
        jnp.concatenate([z_ec, wge, z_ec], axis=1),
    ], axis=0).astype(jnp.bfloat16)
    b3 = jnp.concatenate([g_exc_x_b, g_exc_e_b, w_exc_x_b]).reshape(1, 3 * Ch)

    w7 = jnp.transpose(w_exc_ee_w, (2, 3, 1, 0)).reshape(
        _F, _F * Ch, Ch).astype(jnp.bfloat16)

    row = lambda v: v.reshape(1, Ch)

    def fixed(shape):
        n = len(shape)
        return pl.BlockSpec(shape, lambda b, _n=n: (0,) * _n)

    fn = pl.pallas_call(
        functools.partial(_cell_kernel, H, W, Ch, MT),
        out_shape=jax.ShapeDtypeStruct((N, HW, Ch), jnp.float32),
        grid=(N,),
        in_specs=[
            pl.BlockSpec((1, HW, K3), lambda b: (b, 0, 0)),
            pl.BlockSpec((1, HW, Ch), lambda b: (b, 0, 0)),
            fixed((K3, 3 * Ch)),
            fixed((1, 3 * Ch)),
            fixed((_F, _F * Ch, Ch)),
            fixed((1, Ch)), fixed((1, Ch)),
            fixed((1, Ch)), fixed((1, Ch)),
            fixed((1, Ch)),
            fixed((1, Ch)), fixed((1, Ch)),
        ],
        out_specs=pl.BlockSpec((1, HW, Ch), lambda b: (b, 0, 0)),
        scratch_shapes=[
            pltpu.VMEM((HW + (_F - 1) * W, _F * Ch), jnp.bfloat16),
            pltpu.VMEM((HW, 3 * Ch), jnp.float32),
            pltpu.VMEM((HW, Ch), jnp.float32),
        ],
        compiler_params=pltpu.CompilerParams(
            dimension_semantics=("parallel",)),
    )
    out = fn(
        xe, e_t, w3, b3, w7,
        row(ln_e_x_g), row(ln_e_x_b),
        row(ln_e_e_g), row(ln_e_e_b),
        row(w_exc_ee_b),
        row(ln_out_e_g), row(ln_out_e_b),
    )
    return jnp.transpose(out, (0, 2, 1)).reshape(N, Ch, H, W)
